# baseline (speedup 1.0000x reference)
import functools

import jax
import jax.numpy as jnp
from jax.experimental import pallas as pl
from jax.experimental.pallas import tpu as pltpu
from jax.experimental.pallas import tpu_sc as plsc

D_MODEL = 1024
D_FF = 4096
E = 8
TOP_K = 2
CAP = 1280

N_TOK = 4096
TBLK = 512
N_TB = N_TOK // TBLK
CTILE = 1280
N_CT = CAP // CTILE
S_TOT = E * CAP
DBLK = 256


def _router_kernel(x_ref, g_ref, b_ref, wg_ref,
                   xln_ref, table_ref, sid1_ref, sid2_ref, we1_ref, we2_ref,
                   carry_ref):
    t = pl.program_id(0)

    @pl.when(t == 0)
    def _init():
        carry_ref[...] = jnp.zeros_like(carry_ref)
        table_ref[...] = jnp.zeros_like(table_ref)

    x = x_ref[...]
    mu = jnp.mean(x, axis=1, keepdims=True)
    xc = x - mu
    var = jnp.mean(xc * xc, axis=1, keepdims=True)
    xln = xc * jax.lax.rsqrt(var + 1e-5) * g_ref[...] + b_ref[...]
    xln_ref[...] = xln.astype(jnp.bfloat16)

    logits = jnp.dot(xln, wg_ref[...], preferred_element_type=jnp.float32)
    col = jax.lax.broadcasted_iota(jnp.int32, (TBLK, 128), 1)
    neg = jnp.float32(-1e30)
    logits = jnp.where(col < E, logits, neg)
    v1 = jnp.max(logits, axis=1, keepdims=True)
    i1 = jnp.min(jnp.where(logits == v1, col, 128), axis=1, keepdims=True)
    l2 = jnp.where(col == i1, neg, logits)
    v2 = jnp.max(l2, axis=1, keepdims=True)
    i2 = jnp.min(jnp.where(l2 == v2, col, 128), axis=1, keepdims=True)
    w1 = 1.0 / (1.0 + jnp.exp(v2 - v1))
    w2 = 1.0 - w1

    oh1 = (col == i1).astype(jnp.float32)
    oh2 = (col == i2).astype(jnp.float32)
    ohsum = oh1 + oh2
    ri = jax.lax.broadcasted_iota(jnp.int32, (TBLK, TBLK), 0)
    ci = jax.lax.broadcasted_iota(jnp.int32, (TBLK, TBLK), 1)
    tri = (ci < ri).astype(jnp.bfloat16)
    s_excl = jnp.dot(tri, ohsum.astype(jnp.bfloat16),
                     preferred_element_type=jnp.float32)
    base = carry_ref[...] + s_excl
    pos1 = jnp.round(jnp.sum(oh1 * base, axis=1, keepdims=True))
    pos2 = jnp.round(jnp.sum(oh2 * (base + oh1), axis=1, keepdims=True))
    carry_ref[...] += jnp.sum(ohsum, axis=0, keepdims=True)

    keep1 = (pos1 < CAP).astype(jnp.float32)
    keep2 = (pos2 < CAP).astype(jnp.float32)
    pos1c = jnp.minimum(pos1, CAP - 1)
    pos2c = jnp.minimum(pos2, CAP - 1)

    e1f = jnp.sum(oh1 * col.astype(jnp.float32), axis=1, keepdims=True)
    e2f = jnp.sum(oh2 * col.astype(jnp.float32), axis=1, keepdims=True)
    sid1_ref[...] = (e1f * CAP + pos1c).astype(jnp.int32)
    sid2_ref[...] = (e2f * CAP + pos2c).astype(jnp.int32)
    we1_ref[...] = w1 * keep1
    we2_ref[...] = w2 * keep2

    gtok = (jax.lax.broadcasted_iota(jnp.int32, (TBLK, 1), 0) + t * TBLK) + 1
    thi = (gtok // 64).astype(jnp.float32)
    tlo = (gtok - (gtok // 64) * 64).astype(jnp.float32)
    cdiv2 = col // 2
    csel = col - cdiv2 * 2
    valid_col = col < 2 * E

    def make_vals(i_e, tok_hi, tok_lo):
        eq = (cdiv2 == i_e) & valid_col
        pay = jnp.where(csel == 0, tok_hi, tok_lo)
        return jnp.where(eq, pay, 0.0).astype(jnp.bfloat16)

    vals1 = make_vals(i1, thi, tlo)
    vals2 = make_vals(i2, thi, tlo)

    pr = jax.lax.broadcasted_iota(jnp.int32, (TBLK, CAP), 1)
    p1 = ((pr == pos1c.astype(jnp.int32)) & (keep1 > 0)).astype(jnp.bfloat16)
    p2 = ((pr == pos2c.astype(jnp.int32)) & (keep2 > 0)).astype(jnp.bfloat16)
    dnum = (((0,), (0,)), ((), ()))
    table_ref[...] += (
        jax.lax.dot_general(p1, vals1, dnum, preferred_element_type=jnp.float32)
        + jax.lax.dot_general(p2, vals2, dnum, preferred_element_type=jnp.float32))


def _router_call(x2d, gamma, beta, wg_pad):
    return pl.pallas_call(
        _router_kernel,
        grid=(N_TB,),
        in_specs=[
            pl.BlockSpec((TBLK, D_MODEL), lambda t: (t, 0)),
            pl.BlockSpec((1, D_MODEL), lambda t: (0, 0)),
            pl.BlockSpec((1, D_MODEL), lambda t: (0, 0)),
            pl.BlockSpec((D_MODEL, 128), lambda t: (0, 0)),
        ],
        out_specs=[
            pl.BlockSpec((TBLK, D_MODEL), lambda t: (t, 0)),
            pl.BlockSpec((CAP, 128), lambda t: (0, 0)),
            pl.BlockSpec((TBLK, 1), lambda t: (t, 0)),
            pl.BlockSpec((TBLK, 1), lambda t: (t, 0)),
            pl.BlockSpec((TBLK, 1), lambda t: (t, 0)),
            pl.BlockSpec((TBLK, 1), lambda t: (t, 0)),
        ],
        out_shape=[
            jax.ShapeDtypeStruct((N_TOK, D_MODEL), jnp.bfloat16),
            jax.ShapeDtypeStruct((CAP, 128), jnp.float32),
            jax.ShapeDtypeStruct((N_TOK, 1), jnp.int32),
            jax.ShapeDtypeStruct((N_TOK, 1), jnp.int32),
            jax.ShapeDtypeStruct((N_TOK, 1), jnp.float32),
            jax.ShapeDtypeStruct((N_TOK, 1), jnp.float32),
        ],
        scratch_shapes=[pltpu.VMEM((1, 128), jnp.float32)],
        compiler_params=pltpu.CompilerParams(
            dimension_semantics=("arbitrary",)),
    )(x2d, gamma, beta, wg_pad)


F_SPLIT = 4
D_FH = D_FF // F_SPLIT


def _ffn_kernel(stok_ref, xln_ref, w1_ref, b1_ref, w2_ref, b2_ref, out_ref,
                xe_ref):
    f = pl.program_id(1)

    @pl.when(f == 0)
    def _gather():
        tok = stok_ref[...]
        it = jax.lax.broadcasted_iota(jnp.int32, (CTILE, N_TOK), 1)
        oh = (it == tok).astype(jnp.bfloat16)
        x = jnp.dot(oh, xln_ref[...], preferred_element_type=jnp.float32)
        xe_ref[...] = x.astype(jnp.bfloat16)

    x = xe_ref[...]
    h = jnp.dot(x, w1_ref[0].astype(jnp.bfloat16),
                preferred_element_type=jnp.float32) + b1_ref[0]
    h = jax.nn.gelu(h)
    p = jnp.dot(h.astype(jnp.bfloat16), w2_ref[0].astype(jnp.bfloat16),
                preferred_element_type=jnp.float32)

    @pl.when(f == 0)
    def _first():
        out_ref[...] = (p + b2_ref[0]).astype(jnp.bfloat16)

    @pl.when(f > 0)
    def _rest():
        out_ref[...] = (out_ref[...] + p).astype(jnp.bfloat16)


def _ffn_call(s_tok, xln, w1, b1, w2, b2):
    return pl.pallas_call(
        _ffn_kernel,
        grid=(E, F_SPLIT),
        in_specs=[
            pl.BlockSpec((CTILE, 1), lambda e, f: (e, 0)),
            pl.BlockSpec((N_TOK, D_MODEL), lambda e, f: (0, 0)),
            pl.BlockSpec((1, D_MODEL, D_FH), lambda e, f: (e, 0, f)),
            pl.BlockSpec((1, 1, D_FH), lambda e, f: (e, 0, f)),
            pl.BlockSpec((1, D_FH, D_MODEL), lambda e, f: (e, f, 0)),
            pl.BlockSpec((1, 1, D_MODEL), lambda e, f: (e, 0, 0)),
        ],
        out_specs=pl.BlockSpec((CTILE, D_MODEL), lambda e, f: (e, 0)),
        out_shape=jax.ShapeDtypeStruct((S_TOT, D_MODEL), jnp.bfloat16),
        scratch_shapes=[
            pltpu.VMEM((CAP, D_MODEL), jnp.bfloat16),
        ],
        compiler_params=pltpu.CompilerParams(
            dimension_semantics=("arbitrary", "arbitrary")),
    )(s_tok, xln, w1, b1, w2, b2)


def _combine_kernel(sid1_ref, sid2_ref, we1_ref, we2_ref, oute_ref, data_ref,
                    y_ref):
    si = jax.lax.broadcasted_iota(jnp.int32, (DBLK, S_TOT), 1)
    oh = (jnp.where(si == sid1_ref[...], we1_ref[...], 0.0)
          + jnp.where(si == sid2_ref[...], we2_ref[...], 0.0)).astype(jnp.bfloat16)
    y = jnp.dot(oh, oute_ref[...], preferred_element_type=jnp.float32)
    y_ref[...] = y + data_ref[...]


def _combine_call(sid1, sid2, we1, we2, out_e, data2d):
    nb = N_TOK // DBLK
    return pl.pallas_call(
        _combine_kernel,
        grid=(nb,),
        in_specs=[
            pl.BlockSpec((DBLK, 1), lambda t: (t, 0)),
            pl.BlockSpec((DBLK, 1), lambda t: (t, 0)),
            pl.BlockSpec((DBLK, 1), lambda t: (t, 0)),
            pl.BlockSpec((DBLK, 1), lambda t: (t, 0)),
            pl.BlockSpec((S_TOT, D_MODEL), lambda t: (0, 0)),
            pl.BlockSpec((DBLK, D_MODEL), lambda t: (t, 0)),
        ],
        out_specs=pl.BlockSpec((DBLK, D_MODEL), lambda t: (t, 0)),
        out_shape=jax.ShapeDtypeStruct((N_TOK, D_MODEL), jnp.float32),
        compiler_params=pltpu.CompilerParams(
            dimension_semantics=("arbitrary",)),
    )(sid1, sid2, we1, we2, out_e, data2d)


@jax.jit
def kernel(data, gamma, beta, Wg, W1, b1, W2, b2):
    B, S, D = data.shape
    x2d = data.reshape(B * S, D)
    wg_pad = jnp.zeros((D_MODEL, 128), jnp.float32).at[:, :E].set(Wg)
    g2 = gamma.reshape(1, D_MODEL)
    b2d = beta.reshape(1, D_MODEL)

    xln, table, sid1, sid2, we1, we2 = _router_call(x2d, g2, b2d, wg_pad)

    t_hi = jnp.round(table[:, 0:2 * E:2].T.reshape(S_TOT, 1)).astype(jnp.int32)
    t_lo = jnp.round(table[:, 1:2 * E:2].T.reshape(S_TOT, 1)).astype(jnp.int32)
    s_tok = t_hi * 64 + t_lo - 1

    out_e = _ffn_call(s_tok, xln, W1, b1.reshape(E, 1, D_FF),
                      W2, b2.reshape(E, 1, D_MODEL))

    y = _combine_call(sid1, sid2, we1, we2, out_e, x2d)
    return y.reshape(B, S, D)


# speedup vs baseline: 1.0496x; 1.0496x over previous
import functools

import jax
import jax.numpy as jnp
from jax.experimental import pallas as pl
from jax.experimental.pallas import tpu as pltpu
from jax.experimental.pallas import tpu_sc as plsc

D_MODEL = 1024
D_FF = 4096
E = 8
TOP_K = 2
CAP = 1280

N_TOK = 4096
TBLK = 512
N_TB = N_TOK // TBLK
CTILE = 640
N_CT = CAP // CTILE
S_TOT = E * CAP
DBLK = 256


def _router_kernel(x_ref, g_ref, b_ref, wg_ref,
                   xln_ref, table_ref, sid1_ref, sid2_ref, we1_ref, we2_ref,
                   carry_ref):
    t = pl.program_id(0)

    @pl.when(t == 0)
    def _init():
        carry_ref[...] = jnp.zeros_like(carry_ref)
        table_ref[...] = jnp.zeros_like(table_ref)

    x = x_ref[...]
    mu = jnp.mean(x, axis=1, keepdims=True)
    xc = x - mu
    var = jnp.mean(xc * xc, axis=1, keepdims=True)
    xln = xc * jax.lax.rsqrt(var + 1e-5) * g_ref[...] + b_ref[...]
    xln_ref[...] = xln.astype(jnp.bfloat16)

    logits = jnp.dot(xln, wg_ref[...], preferred_element_type=jnp.float32)
    col = jax.lax.broadcasted_iota(jnp.int32, (TBLK, 128), 1)
    neg = jnp.float32(-1e30)
    logits = jnp.where(col < E, logits, neg)
    v1 = jnp.max(logits, axis=1, keepdims=True)
    i1 = jnp.min(jnp.where(logits == v1, col, 128), axis=1, keepdims=True)
    l2 = jnp.where(col == i1, neg, logits)
    v2 = jnp.max(l2, axis=1, keepdims=True)
    i2 = jnp.min(jnp.where(l2 == v2, col, 128), axis=1, keepdims=True)
    w1 = 1.0 / (1.0 + jnp.exp(v2 - v1))
    w2 = 1.0 - w1

    oh1 = (col == i1).astype(jnp.float32)
    oh2 = (col == i2).astype(jnp.float32)
    ohsum = oh1 + oh2
    ri = jax.lax.broadcasted_iota(jnp.int32, (TBLK, TBLK), 0)
    ci = jax.lax.broadcasted_iota(jnp.int32, (TBLK, TBLK), 1)
    tri = (ci < ri).astype(jnp.bfloat16)
    s_excl = jnp.dot(tri, ohsum.astype(jnp.bfloat16),
                     preferred_element_type=jnp.float32)
    base = carry_ref[...] + s_excl
    pos1 = jnp.round(jnp.sum(oh1 * base, axis=1, keepdims=True))
    pos2 = jnp.round(jnp.sum(oh2 * (base + oh1), axis=1, keepdims=True))
    carry_ref[...] += jnp.sum(ohsum, axis=0, keepdims=True)

    keep1 = (pos1 < CAP).astype(jnp.float32)
    keep2 = (pos2 < CAP).astype(jnp.float32)
    pos1c = jnp.minimum(pos1, CAP - 1)
    pos2c = jnp.minimum(pos2, CAP - 1)

    e1f = jnp.sum(oh1 * col.astype(jnp.float32), axis=1, keepdims=True)
    e2f = jnp.sum(oh2 * col.astype(jnp.float32), axis=1, keepdims=True)
    sid1_ref[...] = (e1f * CAP + pos1c).astype(jnp.int32)
    sid2_ref[...] = (e2f * CAP + pos2c).astype(jnp.int32)
    we1_ref[...] = w1 * keep1
    we2_ref[...] = w2 * keep2

    gtok = (jax.lax.broadcasted_iota(jnp.int32, (TBLK, 1), 0) + t * TBLK) + 1
    thi = (gtok // 64).astype(jnp.float32)
    tlo = (gtok - (gtok // 64) * 64).astype(jnp.float32)
    cdiv2 = col // 2
    csel = col - cdiv2 * 2
    valid_col = col < 2 * E

    def make_vals(i_e, tok_hi, tok_lo):
        eq = (cdiv2 == i_e) & valid_col
        pay = jnp.where(csel == 0, tok_hi, tok_lo)
        return jnp.where(eq, pay, 0.0).astype(jnp.bfloat16)

    vals1 = make_vals(i1, thi, tlo)
    vals2 = make_vals(i2, thi, tlo)

    pr = jax.lax.broadcasted_iota(jnp.int32, (TBLK, CAP), 1)
    p1 = ((pr == pos1c.astype(jnp.int32)) & (keep1 > 0)).astype(jnp.bfloat16)
    p2 = ((pr == pos2c.astype(jnp.int32)) & (keep2 > 0)).astype(jnp.bfloat16)
    dnum = (((0,), (0,)), ((), ()))
    table_ref[...] += (
        jax.lax.dot_general(p1, vals1, dnum, preferred_element_type=jnp.float32)
        + jax.lax.dot_general(p2, vals2, dnum, preferred_element_type=jnp.float32))


def _router_call(x2d, gamma, beta, wg_pad):
    return pl.pallas_call(
        _router_kernel,
        grid=(N_TB,),
        in_specs=[
            pl.BlockSpec((TBLK, D_MODEL), lambda t: (t, 0)),
            pl.BlockSpec((1, D_MODEL), lambda t: (0, 0)),
            pl.BlockSpec((1, D_MODEL), lambda t: (0, 0)),
            pl.BlockSpec((D_MODEL, 128), lambda t: (0, 0)),
        ],
        out_specs=[
            pl.BlockSpec((TBLK, D_MODEL), lambda t: (t, 0)),
            pl.BlockSpec((CAP, 128), lambda t: (0, 0)),
            pl.BlockSpec((TBLK, 1), lambda t: (t, 0)),
            pl.BlockSpec((TBLK, 1), lambda t: (t, 0)),
            pl.BlockSpec((TBLK, 1), lambda t: (t, 0)),
            pl.BlockSpec((TBLK, 1), lambda t: (t, 0)),
        ],
        out_shape=[
            jax.ShapeDtypeStruct((N_TOK, D_MODEL), jnp.bfloat16),
            jax.ShapeDtypeStruct((CAP, 128), jnp.float32),
            jax.ShapeDtypeStruct((N_TOK, 1), jnp.int32),
            jax.ShapeDtypeStruct((N_TOK, 1), jnp.int32),
            jax.ShapeDtypeStruct((N_TOK, 1), jnp.float32),
            jax.ShapeDtypeStruct((N_TOK, 1), jnp.float32),
        ],
        scratch_shapes=[pltpu.VMEM((1, 128), jnp.float32)],
        compiler_params=pltpu.CompilerParams(
            dimension_semantics=("arbitrary",)),
    )(x2d, gamma, beta, wg_pad)


F_SPLIT = 2
D_FH = D_FF // F_SPLIT
NB_S = S_TOT // CTILE


def _ffn_kernel(stok_ref, xln_ref, w1_ref, b1_ref, w2_ref, b2_ref, out_ref,
                xe_ref, oacc_ref):
    f = pl.program_id(1)
    t = pl.program_id(2)
    row0 = t * CTILE

    @pl.when(f == 0)
    def _gather():
        tok = stok_ref[...]
        it = jax.lax.broadcasted_iota(jnp.int32, (CTILE, N_TOK), 1)
        oh = (it == tok).astype(jnp.bfloat16)
        x = jnp.dot(oh, xln_ref[...], preferred_element_type=jnp.float32)
        xe_ref[pl.ds(row0, CTILE), :] = x.astype(jnp.bfloat16)

    x = xe_ref[pl.ds(row0, CTILE), :]
    h = jnp.dot(x, w1_ref[0].astype(jnp.bfloat16),
                preferred_element_type=jnp.float32) + b1_ref[0]
    h = jax.nn.gelu(h)
    p = jnp.dot(h.astype(jnp.bfloat16), w2_ref[0].astype(jnp.bfloat16),
                preferred_element_type=jnp.float32)

    @pl.when(f == 0)
    def _first():
        oacc_ref[pl.ds(row0, CTILE), :] = (p + b2_ref[0]).astype(jnp.bfloat16)
        out_ref[...] = p.astype(jnp.bfloat16)

    @pl.when(f == F_SPLIT - 1)
    def _last():
        out_ref[...] = (oacc_ref[pl.ds(row0, CTILE), :] + p).astype(jnp.bfloat16)


def _ffn_call(s_tok, xln, w1, b1, w2, b2):
    return pl.pallas_call(
        _ffn_kernel,
        grid=(E, F_SPLIT, N_CT),
        in_specs=[
            pl.BlockSpec((CTILE, 1), lambda e, f, t: (e * N_CT + t, 0)),
            pl.BlockSpec((N_TOK, D_MODEL), lambda e, f, t: (0, 0)),
            pl.BlockSpec((1, D_MODEL, D_FH), lambda e, f, t: (e, 0, f)),
            pl.BlockSpec((1, 1, D_FH), lambda e, f, t: (e, 0, f)),
            pl.BlockSpec((1, D_FH, D_MODEL), lambda e, f, t: (e, f, 0)),
            pl.BlockSpec((1, 1, D_MODEL), lambda e, f, t: (e, 0, 0)),
        ],
        out_specs=pl.BlockSpec(
            (CTILE, D_MODEL), lambda e, f, t: (f * NB_S + e * N_CT + t, 0)),
        out_shape=jax.ShapeDtypeStruct((F_SPLIT * S_TOT, D_MODEL), jnp.bfloat16),
        scratch_shapes=[
            pltpu.VMEM((CAP, D_MODEL), jnp.bfloat16),
            pltpu.VMEM((CAP, D_MODEL), jnp.bfloat16),
        ],
        compiler_params=pltpu.CompilerParams(
            dimension_semantics=("arbitrary", "arbitrary", "arbitrary")),
    )(s_tok, xln, w1, b1, w2, b2)


def _combine_kernel(sid1_ref, sid2_ref, we1_ref, we2_ref, oute_ref, data_ref,
                    y_ref):
    si = jax.lax.broadcasted_iota(jnp.int32, (DBLK, S_TOT), 1)
    oh = (jnp.where(si == sid1_ref[...], we1_ref[...], 0.0)
          + jnp.where(si == sid2_ref[...], we2_ref[...], 0.0)).astype(jnp.bfloat16)
    y = jnp.dot(oh, oute_ref[...], preferred_element_type=jnp.float32)
    y_ref[...] = y + data_ref[...]


def _combine_call(sid1, sid2, we1, we2, out_e, data2d):
    nb = N_TOK // DBLK
    return pl.pallas_call(
        _combine_kernel,
        grid=(nb,),
        in_specs=[
            pl.BlockSpec((DBLK, 1), lambda t: (t, 0)),
            pl.BlockSpec((DBLK, 1), lambda t: (t, 0)),
            pl.BlockSpec((DBLK, 1), lambda t: (t, 0)),
            pl.BlockSpec((DBLK, 1), lambda t: (t, 0)),
            pl.BlockSpec((S_TOT, D_MODEL), lambda t: (F_SPLIT - 1, 0)),
            pl.BlockSpec((DBLK, D_MODEL), lambda t: (t, 0)),
        ],
        out_specs=pl.BlockSpec((DBLK, D_MODEL), lambda t: (t, 0)),
        out_shape=jax.ShapeDtypeStruct((N_TOK, D_MODEL), jnp.float32),
        compiler_params=pltpu.CompilerParams(
            dimension_semantics=("arbitrary",)),
    )(sid1, sid2, we1, we2, out_e, data2d)


@jax.jit
def kernel(data, gamma, beta, Wg, W1, b1, W2, b2):
    B, S, D = data.shape
    x2d = data.reshape(B * S, D)
    wg_pad = jnp.zeros((D_MODEL, 128), jnp.float32).at[:, :E].set(Wg)
    g2 = gamma.reshape(1, D_MODEL)
    b2d = beta.reshape(1, D_MODEL)

    xln, table, sid1, sid2, we1, we2 = _router_call(x2d, g2, b2d, wg_pad)

    t_hi = jnp.round(table[:, 0:2 * E:2].T.reshape(S_TOT, 1)).astype(jnp.int32)
    t_lo = jnp.round(table[:, 1:2 * E:2].T.reshape(S_TOT, 1)).astype(jnp.int32)
    s_tok = t_hi * 64 + t_lo - 1

    out_e = _ffn_call(s_tok, xln, W1, b1.reshape(E, 1, D_FF),
                      W2, b2.reshape(E, 1, D_MODEL))

    y = _combine_call(sid1, sid2, we1, we2, out_e, x2d)
    return y.reshape(B, S, D)
